# baseline (speedup 1.0000x reference)
import functools

import jax
import jax.numpy as jnp
from jax.experimental import pallas as pl
from jax.experimental.pallas import tpu as pltpu

_EPS = 1e-5


def _conv_stats_kernel(x_ref, w_ref, xw_ref, s_ref, q_ref, *, bb):
    w = w_ref[...]
    for i in range(bb):
        x_nc = x_ref[i]
        xw = jax.lax.dot_general(
            x_nc.astype(jnp.bfloat16), w,
            (((1,), (0,)), ((), ())),
            preferred_element_type=jnp.float32)
        xw_ref[i] = xw.astype(jnp.bfloat16)
        s_ref[i] = jnp.sum(xw, axis=0, keepdims=True)
        q_ref[i] = jnp.sum(xw * xw, axis=0, keepdims=True)


def _encoding_kernel(xw_ref, a2_ref, b2_ref, cwb_ref, cw_ref, sc_ref,
                     c2_ref, e_ref, s1_ref, q1_ref, *, bb):
    cwb = cwb_ref[...]
    n = xw_ref.shape[1]
    c = xw_ref.shape[2]
    s1 = None
    q1 = None

    xw = xw_ref[...].reshape(bb * n, c).astype(jnp.float32)
    xn = jnp.maximum(xw * a2_ref[...] + b2_ref[...], 0.0)
    xnb = xn.astype(jnp.bfloat16)

    x2 = jnp.sum(xn * xn, axis=-1, keepdims=True)
    xc = jax.lax.dot_general(
        xnb, cwb, (((1,), (1,)), ((), ())),
        preferred_element_type=jnp.float32)
    sl = sc_ref[...] * (x2 - 2.0 * xc + c2_ref[...])

    p = jnp.exp(sl)
    attn = p * pl.reciprocal(jnp.sum(p, axis=-1, keepdims=True) + 1e-30,
                             approx=True)
    attnb = attn.astype(jnp.bfloat16)

    for i in range(bb):
        ax = jax.lax.dot_general(
            attnb[i * n:(i + 1) * n], xnb[i * n:(i + 1) * n],
            (((0,), (0,)), ((), ())),
            preferred_element_type=jnp.float32)
        asum = jnp.sum(attn[i * n:(i + 1) * n], axis=0)
        eb = ax - asum[:, None] * cw_ref[...]
        e_ref[i] = eb
        rs = jnp.sum(eb, axis=1, keepdims=True)
        qs = jnp.sum(eb * eb, axis=1, keepdims=True)
        s1 = rs if s1 is None else s1 + rs
        q1 = qs if q1 is None else q1 + qs
    s1_ref[0] = s1
    q1_ref[0] = q1


def _modulate_kernel(x_ref, g_ref, o_ref):
    o_ref[...] = jnp.maximum(x_ref[...] * (1.0 + g_ref[...]), 0.0)


def kernel(x, conv_w, bn2_w, bn2_b, codewords, scale, bn1_w, bn1_b,
           fc_w, fc_b, se_w, se_b):
    B, C, H, W = x.shape
    N = H * W
    K = codewords.shape[0]
    BB = 4 if B % 4 == 0 else 1
    NB = B // BB
    BS = 8 if B % 8 == 0 else BB
    NS = B // BS

    x_nc = x.transpose(0, 2, 3, 1).reshape(B, N, C)
    w_b = conv_w.T.astype(jnp.bfloat16)

    xw_b, s2, q2 = pl.pallas_call(
        functools.partial(_conv_stats_kernel, bb=BS),
        grid=(NS,),
        in_specs=[
            pl.BlockSpec((BS, N, C), lambda i: (i, 0, 0)),
            pl.BlockSpec((C, C), lambda i: (0, 0)),
        ],
        out_specs=[
            pl.BlockSpec((BS, N, C), lambda i: (i, 0, 0)),
            pl.BlockSpec((BS, 1, C), lambda i: (i, 0, 0)),
            pl.BlockSpec((BS, 1, C), lambda i: (i, 0, 0)),
        ],
        out_shape=[
            jax.ShapeDtypeStruct((B, N, C), jnp.bfloat16),
            jax.ShapeDtypeStruct((B, 1, C), jnp.float32),
            jax.ShapeDtypeStruct((B, 1, C), jnp.float32),
        ],
        compiler_params=pltpu.CompilerParams(
            dimension_semantics=("parallel",),
            vmem_limit_bytes=48 * 1024 * 1024),
    )(x_nc, w_b)

    cnt2 = float(B * N)
    mean2 = jnp.sum(s2[:, 0, :], axis=0) / cnt2
    var2 = jnp.sum(q2[:, 0, :], axis=0) / cnt2 - mean2 * mean2
    a2 = bn2_w * jax.lax.rsqrt(var2 + _EPS)
    b2 = bn2_b - mean2 * a2

    c2_row = jnp.sum(codewords ** 2, axis=1)[None, :]
    cw_b = codewords.astype(jnp.bfloat16)
    e = pl.pallas_call(
        functools.partial(_encoding_kernel, bb=BS),
        grid=(NS,),
        in_specs=[
            pl.BlockSpec((BS, N, C), lambda i: (i, 0, 0)),
            pl.BlockSpec((1, C), lambda i: (0, 0)),
            pl.BlockSpec((1, C), lambda i: (0, 0)),
            pl.BlockSpec((K, C), lambda i: (0, 0)),
            pl.BlockSpec((K, C), lambda i: (0, 0)),
            pl.BlockSpec((1, K), lambda i: (0, 0)),
            pl.BlockSpec((1, K), lambda i: (0, 0)),
        ],
        out_specs=[
            pl.BlockSpec((BS, K, C), lambda i: (i, 0, 0)),
            pl.BlockSpec((1, K, 1), lambda i: (i, 0, 0)),
            pl.BlockSpec((1, K, 1), lambda i: (i, 0, 0)),
        ],
        out_shape=[
            jax.ShapeDtypeStruct((B, K, C), jnp.float32),
            jax.ShapeDtypeStruct((NS, K, 1), jnp.float32),
            jax.ShapeDtypeStruct((NS, K, 1), jnp.float32),
        ],
        compiler_params=pltpu.CompilerParams(
            dimension_semantics=("parallel",),
            vmem_limit_bytes=56 * 1024 * 1024),
    )(xw_b, a2[None, :], b2[None, :], cw_b, codewords, scale[None, :],
      c2_row)
    e, s1, q1 = e

    cnt1 = float(B * C)
    mean1 = jnp.sum(s1[:, :, 0], axis=0) / cnt1
    var1 = jnp.sum(q1[:, :, 0], axis=0) / cnt1 - mean1 * mean1
    a1 = bn1_w * jax.lax.rsqrt(var1 + _EPS)
    b1 = bn1_b - mean1 * a1
    en = jnp.mean(jnp.maximum(e * a1[None, :, None] + b1[None, :, None], 0.0),
                  axis=1)
    hi = jax.lax.Precision.HIGHEST
    gamma = jax.nn.sigmoid(jnp.dot(en, fc_w.T, precision=hi) + fc_b)
    se = jnp.dot(en, se_w.T, precision=hi) + se_b

    out_nc = pl.pallas_call(
        _modulate_kernel,
        grid=(NS,),
        in_specs=[
            pl.BlockSpec((BS, N, C), lambda i: (i, 0, 0)),
            pl.BlockSpec((BS, 1, C), lambda i: (i, 0, 0)),
        ],
        out_specs=pl.BlockSpec((BS, N, C), lambda i: (i, 0, 0)),
        out_shape=jax.ShapeDtypeStruct((B, N, C), jnp.float32),
        compiler_params=pltpu.CompilerParams(
            dimension_semantics=("parallel",),
            vmem_limit_bytes=48 * 1024 * 1024),
    )(x_nc, gamma[:, None, :])

    return out_nc.reshape(B, H, W, C).transpose(0, 3, 1, 2), se


# speedup vs baseline: 1.0182x; 1.0182x over previous
import functools

import jax
import jax.numpy as jnp
from jax.experimental import pallas as pl
from jax.experimental.pallas import tpu as pltpu

_EPS = 1e-5


def _conv_stats_kernel(x_ref, w_ref, xw_ref, s_ref, q_ref, *, bb):
    w = w_ref[...]
    for i in range(bb):
        x_nc = x_ref[i]
        xw = jax.lax.dot_general(
            x_nc.astype(jnp.bfloat16), w,
            (((1,), (0,)), ((), ())),
            preferred_element_type=jnp.float32)
        xw_ref[i] = xw.astype(jnp.bfloat16)
        s_ref[i] = jnp.sum(xw, axis=0, keepdims=True)
        q_ref[i] = jnp.sum(xw * xw, axis=0, keepdims=True)


def _encoding_kernel(xw_ref, a2_ref, b2_ref, cwb_ref, cw_ref, sc_ref,
                     c2_ref, e_ref, *, bb):
    cwb = cwb_ref[...]
    n = xw_ref.shape[1]
    c = xw_ref.shape[2]

    xw = xw_ref[...].reshape(bb * n, c).astype(jnp.float32)
    xn = jnp.maximum(xw * a2_ref[...] + b2_ref[...], 0.0)
    xnb = xn.astype(jnp.bfloat16)

    x2 = jnp.sum(xn * xn, axis=-1, keepdims=True)
    xc = jax.lax.dot_general(
        xnb, cwb, (((1,), (1,)), ((), ())),
        preferred_element_type=jnp.float32)
    sl = sc_ref[...] * (x2 - 2.0 * xc + c2_ref[...])

    p = jnp.exp(sl)
    attn = p * pl.reciprocal(jnp.sum(p, axis=-1, keepdims=True) + 1e-30,
                             approx=True)
    attnb = attn.astype(jnp.bfloat16)

    for i in range(bb):
        ax = jax.lax.dot_general(
            attnb[i * n:(i + 1) * n], xnb[i * n:(i + 1) * n],
            (((0,), (0,)), ((), ())),
            preferred_element_type=jnp.float32)
        asum = jnp.sum(attn[i * n:(i + 1) * n], axis=0)
        e_ref[i] = ax - asum[:, None] * cw_ref[...]


def _modulate_kernel(x_ref, g_ref, o_ref):
    o_ref[...] = jnp.maximum(x_ref[...] * (1.0 + g_ref[...]), 0.0)


def kernel(x, conv_w, bn2_w, bn2_b, codewords, scale, bn1_w, bn1_b,
           fc_w, fc_b, se_w, se_b):
    B, C, H, W = x.shape
    N = H * W
    K = codewords.shape[0]
    BB = 4 if B % 4 == 0 else 1
    NB = B // BB
    BS = 8 if B % 8 == 0 else BB
    NS = B // BS

    x_nc = x.transpose(0, 2, 3, 1).reshape(B, N, C)
    w_b = conv_w.T.astype(jnp.bfloat16)

    xw_b, s2, q2 = pl.pallas_call(
        functools.partial(_conv_stats_kernel, bb=BS),
        grid=(NS,),
        in_specs=[
            pl.BlockSpec((BS, N, C), lambda i: (i, 0, 0)),
            pl.BlockSpec((C, C), lambda i: (0, 0)),
        ],
        out_specs=[
            pl.BlockSpec((BS, N, C), lambda i: (i, 0, 0)),
            pl.BlockSpec((BS, 1, C), lambda i: (i, 0, 0)),
            pl.BlockSpec((BS, 1, C), lambda i: (i, 0, 0)),
        ],
        out_shape=[
            jax.ShapeDtypeStruct((B, N, C), jnp.bfloat16),
            jax.ShapeDtypeStruct((B, 1, C), jnp.float32),
            jax.ShapeDtypeStruct((B, 1, C), jnp.float32),
        ],
        compiler_params=pltpu.CompilerParams(
            dimension_semantics=("parallel",),
            vmem_limit_bytes=48 * 1024 * 1024),
    )(x_nc, w_b)

    cnt2 = float(B * N)
    mean2 = jnp.sum(s2[:, 0, :], axis=0) / cnt2
    var2 = jnp.sum(q2[:, 0, :], axis=0) / cnt2 - mean2 * mean2
    a2 = bn2_w * jax.lax.rsqrt(var2 + _EPS)
    b2 = bn2_b - mean2 * a2

    c2_row = jnp.sum(codewords ** 2, axis=1)[None, :]
    cw_b = codewords.astype(jnp.bfloat16)
    e = pl.pallas_call(
        functools.partial(_encoding_kernel, bb=BS),
        grid=(NS,),
        in_specs=[
            pl.BlockSpec((BS, N, C), lambda i: (i, 0, 0)),
            pl.BlockSpec((1, C), lambda i: (0, 0)),
            pl.BlockSpec((1, C), lambda i: (0, 0)),
            pl.BlockSpec((K, C), lambda i: (0, 0)),
            pl.BlockSpec((K, C), lambda i: (0, 0)),
            pl.BlockSpec((1, K), lambda i: (0, 0)),
            pl.BlockSpec((1, K), lambda i: (0, 0)),
        ],
        out_specs=pl.BlockSpec((BS, K, C), lambda i: (i, 0, 0)),
        out_shape=jax.ShapeDtypeStruct((B, K, C), jnp.float32),
        compiler_params=pltpu.CompilerParams(
            dimension_semantics=("parallel",),
            vmem_limit_bytes=56 * 1024 * 1024),
    )(xw_b, a2[None, :], b2[None, :], cw_b, codewords, scale[None, :],
      c2_row)

    cnt1 = float(B * C)
    mean1 = jnp.sum(e, axis=(0, 2)) / cnt1
    var1 = jnp.sum(e * e, axis=(0, 2)) / cnt1 - mean1 * mean1
    a1 = bn1_w * jax.lax.rsqrt(var1 + _EPS)
    b1 = bn1_b - mean1 * a1
    en = jnp.mean(jnp.maximum(e * a1[None, :, None] + b1[None, :, None], 0.0),
                  axis=1)
    hi = jax.lax.Precision.HIGHEST
    gamma = jax.nn.sigmoid(jnp.dot(en, fc_w.T, precision=hi) + fc_b)
    se = jnp.dot(en, se_w.T, precision=hi) + se_b

    out_nc = pl.pallas_call(
        _modulate_kernel,
        grid=(NS,),
        in_specs=[
            pl.BlockSpec((BS, N, C), lambda i: (i, 0, 0)),
            pl.BlockSpec((BS, 1, C), lambda i: (i, 0, 0)),
        ],
        out_specs=pl.BlockSpec((BS, N, C), lambda i: (i, 0, 0)),
        out_shape=jax.ShapeDtypeStruct((B, N, C), jnp.float32),
        compiler_params=pltpu.CompilerParams(
            dimension_semantics=("parallel",),
            vmem_limit_bytes=48 * 1024 * 1024),
    )(x_nc, gamma[:, None, :])

    return out_nc.reshape(B, H, W, C).transpose(0, 3, 1, 2), se
